# baseline (speedup 1.0000x reference)
import functools

import jax
import jax.numpy as jnp
from jax import lax
from jax.experimental import pallas as pl
from jax.experimental.pallas import tpu as pltpu
from jax.experimental.pallas import tpu_sc as plsc

N = 10000
E = 320000
D_IN = 128
H = 128
C = 64

NC = 2
NS = 16
NW = NC * NS
CHUNK = 128
NCHUNK = E // CHUNK
CH_PER_TILE = 80
CH_HALF = 40
E_PER_TILE = E // NW
N_PAD = 10240
ROWS_PER_TILE = N_PAD // NS
ECH_PAD = NW * CH_PER_TILE


@functools.lru_cache(maxsize=None)
def _mesh():
  return plsc.VectorSubcoreMesh(
      core_axis_name="c", subcore_axis_name="s", num_cores=NC, num_subcores=NS)


@functools.lru_cache(maxsize=None)
def _make_deg_kernel():
  @functools.partial(
      pl.kernel,
      out_type=jax.ShapeDtypeStruct((NC, NS, 2, N), jnp.float32),
      mesh=_mesh(),
      compiler_params=pltpu.CompilerParams(needs_layout_passes=False),
      scratch_types=[
          pltpu.VMEM((CH_PER_TILE, CHUNK), jnp.int32),
          pltpu.VMEM((CH_PER_TILE, CHUNK), jnp.int32),
          pltpu.VMEM((N,), jnp.float32),
          pltpu.VMEM((N,), jnp.float32),
      ],
  )
  def deg_kernel(ei2_hbm, out_hbm, sidx2, didx2, cnt_out, cnt_in):
    c = lax.axis_index("c")
    s = lax.axis_index("s")
    w = c * NS + s
    zeros = jnp.zeros((16,), jnp.float32)

    def zbody(k, carry):
      cnt_out[pl.ds(k * 16, 16)] = zeros
      cnt_in[pl.ds(k * 16, 16)] = zeros
      return carry

    lax.fori_loop(0, N // 16, zbody, 0)
    ch0 = pl.multiple_of(w * CH_PER_TILE, 8)
    nj = jnp.minimum(CH_PER_TILE, jnp.maximum(NCHUNK - w * CH_PER_TILE, 0))
    pltpu.sync_copy(ei2_hbm.at[0, pl.ds(ch0, CH_PER_TILE)], sidx2)
    pltpu.sync_copy(ei2_hbm.at[1, pl.ds(ch0, CH_PER_TILE)], didx2)
    ones = jnp.ones((16,), jnp.float32)

    def body(r, carry):
      for kc in range(CHUNK // 16):
        iv = sidx2[r, pl.ds(kc * 16, 16)]
        plsc.addupdate_scatter(cnt_out, [iv], ones)
        jv = didx2[r, pl.ds(kc * 16, 16)]
        plsc.addupdate_scatter(cnt_in, [jv], ones)
      return carry

    lax.fori_loop(0, nj, body, 0)
    pltpu.sync_copy(cnt_out, out_hbm.at[c, s, 0])
    pltpu.sync_copy(cnt_in, out_hbm.at[c, s, 1])

  return deg_kernel


@functools.lru_cache(maxsize=None)
def _make_mp_kernel(d):
  @functools.partial(
      pl.kernel,
      out_type=jax.ShapeDtypeStruct((NC, N_PAD, d), jnp.float32),
      mesh=_mesh(),
      compiler_params=pltpu.CompilerParams(needs_layout_passes=False),
      scratch_types=[
          pltpu.VMEM((CH_HALF, CHUNK), jnp.int32),
          pltpu.VMEM((CH_HALF, CHUNK), jnp.int32),
          pltpu.VMEM((CHUNK, d), jnp.float32),
          pltpu.VMEM((CHUNK, d), jnp.float32),
          pltpu.VMEM_SHARED((N_PAD, d), jnp.float32),
          pltpu.SemaphoreType.DMA,
          pltpu.SemaphoreType.DMA,
      ],
  )
  def mp_kernel(x_hbm, ei2_hbm, out_hbm,
                sidx, didx, rows0, rows1, acc, sem0, sem1):
    c = lax.axis_index("c")
    s = lax.axis_index("s")
    w = c * NS + s
    r0 = pl.multiple_of(s * ROWS_PER_TILE, 8)
    zeros = jnp.zeros((16,), jnp.float32)

    def zbody(k, carry):
      for kc in range(d // 16):
        rows0[k, pl.ds(kc * 16, 16)] = zeros
      return carry

    lax.fori_loop(0, CHUNK, zbody, 0)
    for zz in range(ROWS_PER_TILE // CHUNK):
      pltpu.sync_copy(rows0, acc.at[pl.ds(r0 + zz * CHUNK, CHUNK)])
    ch0 = pl.multiple_of(w * CH_PER_TILE, 8)
    nj = jnp.minimum(CH_PER_TILE, jnp.maximum(NCHUNK - w * CH_PER_TILE, 0))
    plsc.subcore_barrier()

    for hh in range(CH_PER_TILE // CH_HALF):
      base = hh * CH_HALF
      nj_h = jnp.clip(nj - base, 0, CH_HALF)

      @pl.when(nj_h > 0)
      def _():
        bo = pl.multiple_of(ch0 + base, 8)
        pltpu.sync_copy(ei2_hbm.at[0, pl.ds(bo, CH_HALF)], sidx)
        pltpu.sync_copy(ei2_hbm.at[1, pl.ds(bo, CH_HALF)], didx)
        pltpu.async_copy(x_hbm.at[sidx.at[0]], rows0, sem0)

        def body(g, carry):
          j0 = 2 * g
          j1 = j0 + 1
          pltpu.make_async_copy(x_hbm.at[sidx.at[j0]], rows0, sem0).wait()
          pltpu.async_copy(x_hbm.at[sidx.at[j1]], rows1, sem1)
          pltpu.sync_copy(rows0, acc.at[didx.at[j0]], add=True)
          pltpu.make_async_copy(x_hbm.at[sidx.at[j1]], rows1, sem1).wait()

          @pl.when(j1 + 1 < nj_h)
          def _():
            pltpu.async_copy(x_hbm.at[sidx.at[j1 + 1]], rows0, sem0)

          pltpu.sync_copy(rows1, acc.at[didx.at[j1]], add=True)
          return carry

        lax.fori_loop(0, nj_h // 2, body, 0)
    plsc.subcore_barrier()
    pltpu.sync_copy(acc.at[pl.ds(r0, ROWS_PER_TILE)],
                    out_hbm.at[c, pl.ds(r0, ROWS_PER_TILE)])

  return mp_kernel


BN = 512
_GRID = pl.cdiv(N, BN)


_DEG_SPEC = pl.BlockSpec((NC, NS, 2, BN), lambda i: (0, 0, 0, i))
_SC_SPEC = pl.BlockSpec((2, BN), lambda i: (0, i))


def _tc1_body(feat_b, w1_b, deg_b, xw_b, sc_b):
  sums = jnp.sum(deg_b[...], axis=(0, 1))
  do = lax.rsqrt(jnp.maximum(sums[0], 1.0))
  di = lax.rsqrt(jnp.maximum(sums[1], 1.0))
  sc_b[0, :] = do
  sc_b[1, :] = di
  acc = jnp.dot(feat_b[...], w1_b[...], preferred_element_type=jnp.float32)
  xw_b[...] = acc * do[:, None]


def _tc1(feat, W1, degp):
  return pl.pallas_call(
      _tc1_body,
      out_shape=(jax.ShapeDtypeStruct((N, H), jnp.float32),
                 jax.ShapeDtypeStruct((2, N), jnp.float32)),
      grid=(_GRID,),
      in_specs=[
          pl.BlockSpec((BN, D_IN), lambda i: (i, 0)),
          pl.BlockSpec((D_IN, H), lambda i: (0, 0)),
          _DEG_SPEC,
      ],
      out_specs=(pl.BlockSpec((BN, H), lambda i: (i, 0)), _SC_SPEC),
  )(feat, W1, degp)


def _tc2_body(s1_b, sc_b, b1_b, h1_b, g1_b):
  do = sc_b[0, :]
  di = sc_b[1, :]
  agg = s1_b[0] + s1_b[1]
  h1 = jnp.maximum(agg * di[:, None] + b1_b[...], 0.0)
  h1_b[...] = h1
  g1_b[...] = h1 * do[:, None]


def _tc2(S1, scales, b1r):
  return pl.pallas_call(
      _tc2_body,
      out_shape=(jax.ShapeDtypeStruct((N, H), jnp.float32),
                 jax.ShapeDtypeStruct((N, H), jnp.float32)),
      grid=(_GRID,),
      in_specs=[
          pl.BlockSpec((NC, BN, H), lambda i: (0, i, 0)),
          _SC_SPEC,
          pl.BlockSpec((1, H), lambda i: (0, 0)),
      ],
      out_specs=(pl.BlockSpec((BN, H), lambda i: (i, 0)),
                 pl.BlockSpec((BN, H), lambda i: (i, 0))),
  )(S1, scales, b1r)


def _tc3_body(s2_b, sc_b, w2t_b, b2_b, h2t_b):
  di = sc_b[1, :]
  agg = s2_b[0] + s2_b[1]
  y = lax.dot_general(agg, w2t_b[...], (((1,), (1,)), ((), ())),
                      preferred_element_type=jnp.float32)
  h2 = jnp.maximum(y * di[:, None] + b2_b[...], 0.0)
  h2t_b[...] = h2.T


def _tc3(S2, scales, W2t, b2r):
  return pl.pallas_call(
      _tc3_body,
      out_shape=jax.ShapeDtypeStruct((C, N), jnp.float32),
      grid=(_GRID,),
      in_specs=[
          pl.BlockSpec((NC, BN, H), lambda i: (0, i, 0)),
          _SC_SPEC,
          pl.BlockSpec((C, H), lambda i: (0, 0)),
          pl.BlockSpec((1, C), lambda i: (0, 0)),
      ],
      out_specs=pl.BlockSpec((C, BN), lambda i: (0, i)),
  )(S2, scales, W2t, b2r)


def kernel(feat, edge_index, W1, b1, W2, b2):
  ei2 = jnp.pad(edge_index.reshape(2, NCHUNK, CHUNK),
                ((0, 0), (0, ECH_PAD - NCHUNK), (0, 0)))
  b1r = b1.reshape(1, H)
  b2r = b2.reshape(1, C)

  degp = _make_deg_kernel()(ei2)
  xw1, scales = _tc1(feat, W1, degp)
  mp = _make_mp_kernel(H)
  S1 = mp(xw1, ei2)
  h1, g1 = _tc2(S1, scales, b1r)
  S2 = mp(g1, ei2)
  h2t = _tc3(S2, scales, W2.T, b2r)
  return h1, h2t.T


# speedup vs baseline: 1.0511x; 1.0511x over previous
import functools

import jax
import jax.numpy as jnp
from jax import lax
from jax.experimental import pallas as pl
from jax.experimental.pallas import tpu as pltpu
from jax.experimental.pallas import tpu_sc as plsc

N = 10000
E = 320000
D_IN = 128
H = 128
C = 64

NC = 2
NS = 16
NW = NC * NS
CHUNK = 128
NCHUNK = E // CHUNK
CH_PER_TILE = 80
CH_HALF = 40
E_PER_TILE = E // NW
N_PAD = 10240
ROWS_PER_TILE = N_PAD // NS
ECH_PAD = NW * CH_PER_TILE


@functools.lru_cache(maxsize=None)
def _mesh():
  return plsc.VectorSubcoreMesh(
      core_axis_name="c", subcore_axis_name="s", num_cores=NC, num_subcores=NS)


@functools.lru_cache(maxsize=None)
def _make_deg_kernel():
  @functools.partial(
      pl.kernel,
      out_type=jax.ShapeDtypeStruct((NC, NS, 2, N), jnp.float32),
      mesh=_mesh(),
      compiler_params=pltpu.CompilerParams(needs_layout_passes=False),
      scratch_types=[
          pltpu.VMEM((CH_PER_TILE, CHUNK), jnp.int32),
          pltpu.VMEM((CH_PER_TILE, CHUNK), jnp.int32),
          pltpu.VMEM((N,), jnp.float32),
          pltpu.VMEM((N,), jnp.float32),
          pltpu.SemaphoreType.DMA,
          pltpu.SemaphoreType.DMA,
      ],
  )
  def deg_kernel(ei2_hbm, out_hbm, sidx2, didx2, cnt_out, cnt_in,
                 dsem0, dsem1):
    c = lax.axis_index("c")
    s = lax.axis_index("s")
    w = c * NS + s
    ch0 = pl.multiple_of(w * CH_PER_TILE, 8)
    nj = jnp.minimum(CH_PER_TILE, jnp.maximum(NCHUNK - w * CH_PER_TILE, 0))
    pltpu.async_copy(ei2_hbm.at[0, pl.ds(ch0, CH_PER_TILE)], sidx2, dsem0)
    pltpu.async_copy(ei2_hbm.at[1, pl.ds(ch0, CH_PER_TILE)], didx2, dsem1)
    zeros = jnp.zeros((16,), jnp.float32)

    def zbody(k, carry):
      cnt_out[pl.ds(k * 16, 16)] = zeros
      cnt_in[pl.ds(k * 16, 16)] = zeros
      return carry

    lax.fori_loop(0, N // 16, zbody, 0)
    pltpu.make_async_copy(
        ei2_hbm.at[0, pl.ds(ch0, CH_PER_TILE)], sidx2, dsem0).wait()
    pltpu.make_async_copy(
        ei2_hbm.at[1, pl.ds(ch0, CH_PER_TILE)], didx2, dsem1).wait()
    ones = jnp.ones((16,), jnp.float32)

    def body(r, carry):
      for kc in range(CHUNK // 16):
        iv = sidx2[r, pl.ds(kc * 16, 16)]
        plsc.addupdate_scatter(cnt_out, [iv], ones)
        jv = didx2[r, pl.ds(kc * 16, 16)]
        plsc.addupdate_scatter(cnt_in, [jv], ones)
      return carry

    lax.fori_loop(0, nj, body, 0)
    pltpu.sync_copy(cnt_out, out_hbm.at[c, s, 0])
    pltpu.sync_copy(cnt_in, out_hbm.at[c, s, 1])

  return deg_kernel


@functools.lru_cache(maxsize=None)
def _make_mp_kernel(d):
  @functools.partial(
      pl.kernel,
      out_type=jax.ShapeDtypeStruct((NC, N_PAD, d), jnp.float32),
      mesh=_mesh(),
      compiler_params=pltpu.CompilerParams(needs_layout_passes=False),
      scratch_types=[
          pltpu.VMEM((CH_HALF, CHUNK), jnp.int32),
          pltpu.VMEM((CH_HALF, CHUNK), jnp.int32),
          pltpu.VMEM((CHUNK, d), jnp.float32),
          pltpu.VMEM((CHUNK, d), jnp.float32),
          pltpu.VMEM_SHARED((N_PAD, d), jnp.float32),
          pltpu.SemaphoreType.DMA,
          pltpu.SemaphoreType.DMA,
      ],
  )
  def mp_kernel(x_hbm, ei2_hbm, out_hbm,
                sidx, didx, rows0, rows1, acc, sem0, sem1):
    c = lax.axis_index("c")
    s = lax.axis_index("s")
    w = c * NS + s
    r0 = pl.multiple_of(s * ROWS_PER_TILE, 8)
    zeros = jnp.zeros((16,), jnp.float32)

    def zbody(k, carry):
      for kc in range(d // 16):
        rows0[k, pl.ds(kc * 16, 16)] = zeros
      return carry

    lax.fori_loop(0, CHUNK, zbody, 0)
    for zz in range(ROWS_PER_TILE // CHUNK):
      pltpu.sync_copy(rows0, acc.at[pl.ds(r0 + zz * CHUNK, CHUNK)])
    ch0 = pl.multiple_of(w * CH_PER_TILE, 8)
    nj = jnp.minimum(CH_PER_TILE, jnp.maximum(NCHUNK - w * CH_PER_TILE, 0))
    plsc.subcore_barrier()

    for hh in range(CH_PER_TILE // CH_HALF):
      base = hh * CH_HALF
      nj_h = jnp.clip(nj - base, 0, CH_HALF)

      @pl.when(nj_h > 0)
      def _():
        bo = pl.multiple_of(ch0 + base, 8)
        pltpu.sync_copy(ei2_hbm.at[0, pl.ds(bo, CH_HALF)], sidx)
        pltpu.sync_copy(ei2_hbm.at[1, pl.ds(bo, CH_HALF)], didx)
        pltpu.async_copy(x_hbm.at[sidx.at[0]], rows0, sem0)

        def body(g, carry):
          j0 = 2 * g
          j1 = j0 + 1
          pltpu.make_async_copy(x_hbm.at[sidx.at[j0]], rows0, sem0).wait()
          pltpu.async_copy(x_hbm.at[sidx.at[j1]], rows1, sem1)
          pltpu.sync_copy(rows0, acc.at[didx.at[j0]], add=True)
          pltpu.make_async_copy(x_hbm.at[sidx.at[j1]], rows1, sem1).wait()

          @pl.when(j1 + 1 < nj_h)
          def _():
            pltpu.async_copy(x_hbm.at[sidx.at[j1 + 1]], rows0, sem0)

          pltpu.sync_copy(rows1, acc.at[didx.at[j1]], add=True)
          return carry

        lax.fori_loop(0, nj_h // 2, body, 0)
    plsc.subcore_barrier()
    pltpu.sync_copy(acc.at[pl.ds(r0, ROWS_PER_TILE)],
                    out_hbm.at[c, pl.ds(r0, ROWS_PER_TILE)])

  return mp_kernel


BN = 1024
_GRID = pl.cdiv(N, BN)


_DEG_SPEC = pl.BlockSpec((NC, NS, 2, BN), lambda i: (0, 0, 0, i))
_SC_SPEC = pl.BlockSpec((2, BN), lambda i: (0, i))


def _tc1_body(feat_b, w1_b, deg_b, xw_b, sc_b):
  sums = jnp.sum(deg_b[...], axis=(0, 1))
  do = lax.rsqrt(jnp.maximum(sums[0], 1.0))
  di = lax.rsqrt(jnp.maximum(sums[1], 1.0))
  sc_b[0, :] = do
  sc_b[1, :] = di
  acc = jnp.dot(feat_b[...], w1_b[...], preferred_element_type=jnp.float32)
  xw_b[...] = acc * do[:, None]


def _tc1(feat, W1, degp):
  return pl.pallas_call(
      _tc1_body,
      out_shape=(jax.ShapeDtypeStruct((N, H), jnp.float32),
                 jax.ShapeDtypeStruct((2, N), jnp.float32)),
      grid=(_GRID,),
      in_specs=[
          pl.BlockSpec((BN, D_IN), lambda i: (i, 0)),
          pl.BlockSpec((D_IN, H), lambda i: (0, 0)),
          _DEG_SPEC,
      ],
      out_specs=(pl.BlockSpec((BN, H), lambda i: (i, 0)), _SC_SPEC),
  )(feat, W1, degp)


def _tc2_body(s1_b, sc_b, b1_b, h1_b, g1_b):
  do = sc_b[0, :]
  di = sc_b[1, :]
  agg = s1_b[0] + s1_b[1]
  h1 = jnp.maximum(agg * di[:, None] + b1_b[...], 0.0)
  h1_b[...] = h1
  g1_b[...] = h1 * do[:, None]


def _tc2(S1, scales, b1r):
  return pl.pallas_call(
      _tc2_body,
      out_shape=(jax.ShapeDtypeStruct((N, H), jnp.float32),
                 jax.ShapeDtypeStruct((N, H), jnp.float32)),
      grid=(_GRID,),
      in_specs=[
          pl.BlockSpec((NC, BN, H), lambda i: (0, i, 0)),
          _SC_SPEC,
          pl.BlockSpec((1, H), lambda i: (0, 0)),
      ],
      out_specs=(pl.BlockSpec((BN, H), lambda i: (i, 0)),
                 pl.BlockSpec((BN, H), lambda i: (i, 0))),
  )(S1, scales, b1r)


def _tc3_body(s2_b, sc_b, w2t_b, b2_b, h2t_b):
  di = sc_b[1, :]
  agg = s2_b[0] + s2_b[1]
  y = lax.dot_general(agg, w2t_b[...], (((1,), (1,)), ((), ())),
                      preferred_element_type=jnp.float32)
  h2 = jnp.maximum(y * di[:, None] + b2_b[...], 0.0)
  h2t_b[...] = h2.T


def _tc3(S2, scales, W2t, b2r):
  return pl.pallas_call(
      _tc3_body,
      out_shape=jax.ShapeDtypeStruct((C, N), jnp.float32),
      grid=(_GRID,),
      in_specs=[
          pl.BlockSpec((NC, BN, H), lambda i: (0, i, 0)),
          _SC_SPEC,
          pl.BlockSpec((C, H), lambda i: (0, 0)),
          pl.BlockSpec((1, C), lambda i: (0, 0)),
      ],
      out_specs=pl.BlockSpec((C, BN), lambda i: (0, i)),
  )(S2, scales, W2t, b2r)


def kernel(feat, edge_index, W1, b1, W2, b2):
  ei2 = jnp.pad(edge_index.reshape(2, NCHUNK, CHUNK),
                ((0, 0), (0, ECH_PAD - NCHUNK), (0, 0)))
  b1r = b1.reshape(1, H)
  b2r = b2.reshape(1, C)

  degp = _make_deg_kernel()(ei2)
  xw1, scales = _tc1(feat, W1, degp)
  mp = _make_mp_kernel(H)
  S1 = mp(xw1, ei2)
  h1, g1 = _tc2(S1, scales, b1r)
  S2 = mp(g1, ei2)
  h2t = _tc3(S2, scales, W2.T, b2r)
  return h1, h2t.T
